# baseline (speedup 1.0000x reference)
import functools

import jax
import jax.numpy as jnp
from jax.experimental import pallas as pl
from jax.experimental.pallas import tpu as pltpu

E = 8
D = 2048
H = 4096
T = 8192

TM = 1024
CH = 512


def _mlp_body(x_ref, w1_ref, b1_ref, w2_ref, b2_ref, o_ref):
    for k in range(H // CH):
        sl = slice(k * CH, (k + 1) * CH)
        hk = jnp.dot(x_ref[...], w1_ref[0, :, sl],
                     preferred_element_type=jnp.float32)
        hk = jnp.maximum(hk + b1_ref[0, :, sl], 0.0).astype(jnp.bfloat16)
        pk = jnp.dot(hk, w2_ref[0, sl, :], preferred_element_type=jnp.float32)
        if k == 0:
            o_ref[0] = pk + b2_ref[0]
        else:
            o_ref[0] += pk


@functools.partial(jax.jit, static_argnames=("interpret",))
def kernel(x, W1, b1, W2, b2, interpret=False):
    e, d, h, t = W1.shape[0], x.shape[1], W1.shape[2], x.shape[0]
    xb = x.astype(jnp.bfloat16)
    W1b = W1.astype(jnp.bfloat16)
    W2b = W2.astype(jnp.bfloat16)
    b1r = b1.reshape(e, 1, h)
    b2r = b2.reshape(e, 1, d)
    grid = (e, t // TM)
    single = pl.Buffered(buffer_count=1)
    return pl.pallas_call(
        _mlp_body,
        grid=grid,
        in_specs=[
            pl.BlockSpec((TM, d), lambda ei, ti: (ti, 0), pipeline_mode=single),
            pl.BlockSpec((1, d, h), lambda ei, ti: (ei, 0, 0), pipeline_mode=single),
            pl.BlockSpec((1, 1, h), lambda ei, ti: (ei, 0, 0)),
            pl.BlockSpec((1, h, d), lambda ei, ti: (ei, 0, 0), pipeline_mode=single),
            pl.BlockSpec((1, 1, d), lambda ei, ti: (ei, 0, 0)),
        ],
        out_specs=pl.BlockSpec((1, TM, d), lambda ei, ti: (ei, ti, 0)),
        out_shape=jax.ShapeDtypeStruct((e, t, d), jnp.float32),
        compiler_params=pltpu.CompilerParams(
            dimension_semantics=("arbitrary", "arbitrary"),
            vmem_limit_bytes=100 * 1024 * 1024,
        ),
        interpret=interpret,
    )(xb, W1b, b1r, W2b, b2r)


# speedup vs baseline: 1.5130x; 1.5130x over previous
import functools

import jax
import jax.numpy as jnp
import numpy as np
from jax.experimental import pallas as pl
from jax.experimental.pallas import tpu as pltpu
from jax.sharding import Mesh, PartitionSpec as P

try:
    from jax import shard_map as _shard_map
except ImportError:
    from jax.experimental.shard_map import shard_map as _shard_map

E = 8
D = 2048
H = 4096
T = 8192

TM = 512
CH = 1024


def _mlp_body(x_ref, w1_ref, b1_ref, w2_ref, b2_ref, o_ref):
    x = x_ref[...]
    acc = None
    for k in range(H // CH):
        sl = slice(k * CH, (k + 1) * CH)
        hk = jnp.dot(x, w1_ref[0, :, sl], preferred_element_type=jnp.float32)
        hk = jnp.maximum(hk + b1_ref[0, :, sl], 0.0).astype(jnp.bfloat16)
        pk = jnp.dot(hk, w2_ref[0, sl, :], preferred_element_type=jnp.float32)
        acc = pk if acc is None else acc + pk
    o_ref[0] = acc + b2_ref[0]


def _experts_mlp(xb, W1b, b1r, W2b, b2r, interpret=False):
    e, _, h = W1b.shape
    t, d = xb.shape
    grid = (e, t // TM)
    single = pl.Buffered(buffer_count=1)
    return pl.pallas_call(
        _mlp_body,
        grid=grid,
        in_specs=[
            pl.BlockSpec((TM, d), lambda ei, ti: (ti, 0)),
            pl.BlockSpec((1, d, h), lambda ei, ti: (ei, 0, 0), pipeline_mode=single),
            pl.BlockSpec((1, 1, h), lambda ei, ti: (ei, 0, 0)),
            pl.BlockSpec((1, h, d), lambda ei, ti: (ei, 0, 0), pipeline_mode=single),
            pl.BlockSpec((1, 1, d), lambda ei, ti: (ei, 0, 0)),
        ],
        out_specs=pl.BlockSpec((1, TM, d), lambda ei, ti: (ei, ti, 0)),
        out_shape=jax.ShapeDtypeStruct((e, t, d), jnp.float32),
        compiler_params=pltpu.CompilerParams(
            dimension_semantics=("arbitrary", "arbitrary"),
            vmem_limit_bytes=100 * 1024 * 1024,
        ),
        interpret=interpret,
    )(xb, W1b, b1r, W2b, b2r)


@functools.partial(jax.jit, static_argnames=("interpret",))
def kernel(x, W1, b1, W2, b2, interpret=False):
    e, d, h = W1.shape
    xb = x.astype(jnp.bfloat16)
    W1b = W1.astype(jnp.bfloat16)
    W2b = W2.astype(jnp.bfloat16)
    b1r = b1.reshape(e, 1, h)
    b2r = b2.reshape(e, 1, d)

    devs = jax.devices()
    n_shards = 2 if (len(devs) >= 2 and e % 2 == 0) else 1
    if n_shards == 1 or interpret:
        return _experts_mlp(xb, W1b, b1r, W2b, b2r, interpret=interpret)

    mesh = Mesh(np.array(devs[:n_shards]), ("expert",))
    fn = _shard_map(
        _experts_mlp,
        mesh=mesh,
        in_specs=(P(), P("expert"), P("expert"), P("expert"), P("expert")),
        out_specs=P("expert"),
        check_vma=False,
    )
    return fn(xb, W1b, b1r, W2b, b2r)


# speedup vs baseline: 1.6090x; 1.0635x over previous
import functools

import jax
import jax.numpy as jnp
import numpy as np
from jax.experimental import pallas as pl
from jax.experimental.pallas import tpu as pltpu
from jax.sharding import Mesh, PartitionSpec as P

try:
    from jax import shard_map as _shard_map
except ImportError:
    from jax.experimental.shard_map import shard_map as _shard_map

E = 8
D = 2048
H = 4096
T = 8192

TM = 512
CH = 1024


def _mlp_body(x_ref, w1_ref, b1_ref, w2_ref, b2_ref, o_ref):
    x = x_ref[...]
    acc = None
    for k in range(H // CH):
        sl = slice(k * CH, (k + 1) * CH)
        hk = jnp.dot(x, w1_ref[0, :, sl], preferred_element_type=jnp.float32)
        hk = jnp.maximum(hk + b1_ref[0, :, sl], 0.0).astype(jnp.bfloat16)
        pk = jnp.dot(hk, w2_ref[0, sl, :], preferred_element_type=jnp.float32)
        acc = pk if acc is None else acc + pk
    o_ref[0] = acc + b2_ref[0]


def _experts_mlp(xb, W1, b1r, W2, b2r, interpret=False):
    W1b = W1.astype(jnp.bfloat16)
    W2b = W2.astype(jnp.bfloat16)
    e, _, h = W1b.shape
    t, d = xb.shape
    grid = (e, t // TM)
    single = pl.Buffered(buffer_count=1)
    return pl.pallas_call(
        _mlp_body,
        grid=grid,
        in_specs=[
            pl.BlockSpec((TM, d), lambda ei, ti: (ti, 0)),
            pl.BlockSpec((1, d, h), lambda ei, ti: (ei, 0, 0), pipeline_mode=single),
            pl.BlockSpec((1, 1, h), lambda ei, ti: (ei, 0, 0)),
            pl.BlockSpec((1, h, d), lambda ei, ti: (ei, 0, 0), pipeline_mode=single),
            pl.BlockSpec((1, 1, d), lambda ei, ti: (ei, 0, 0)),
        ],
        out_specs=pl.BlockSpec((1, TM, d), lambda ei, ti: (ei, ti, 0)),
        out_shape=jax.ShapeDtypeStruct((e, t, d), jnp.float32),
        compiler_params=pltpu.CompilerParams(
            dimension_semantics=("arbitrary", "arbitrary"),
            vmem_limit_bytes=100 * 1024 * 1024,
        ),
        interpret=interpret,
    )(xb, W1b, b1r, W2b, b2r)


@functools.partial(jax.jit, static_argnames=("interpret",))
def kernel(x, W1, b1, W2, b2, interpret=False):
    e, d, h = W1.shape
    xb = x.astype(jnp.bfloat16)
    b1r = b1.reshape(e, 1, h)
    b2r = b2.reshape(e, 1, d)

    devs = jax.devices()
    n_shards = 2 if (len(devs) >= 2 and e % 2 == 0) else 1
    if n_shards == 1 or interpret:
        return _experts_mlp(xb, W1, b1r, W2, b2r, interpret=interpret)

    mesh = Mesh(np.array(devs[:n_shards]), ("expert",))
    fn = _shard_map(
        _experts_mlp,
        mesh=mesh,
        in_specs=(P(), P("expert"), P("expert"), P("expert"), P("expert")),
        out_specs=P("expert"),
        check_vma=False,
    )
    return fn(xb, W1, b1r, W2, b2r)
